# baseline (speedup 1.0000x reference)
import functools
import math

import jax
import jax.numpy as jnp
import numpy as np
from jax import lax
from jax.experimental import pallas as pl
from jax.experimental.pallas import tpu as pltpu
from jax.experimental.pallas import tpu_sc as plsc

D_MODEL = 64
VOCAB = 1000000
MAX_LEN = 512
BATCH = 1024
SEQ = 200

NC = 2
NS = 16
NW = NC * NS

ROWS = BATCH * SEQ
ROWS_PER_W = ROWS // NW
CHUNK = SEQ
NCHUNK = ROWS_PER_W // CHUNK
HALF = CHUNK // 2


def _make_pe():
    position = np.arange(0, SEQ, dtype=np.float32)[:, None]
    div_term = np.exp(
        np.arange(0, D_MODEL, 2, dtype=np.float32) * -(math.log(10000.0) / D_MODEL)
    )
    pe = np.zeros((SEQ, D_MODEL), dtype=np.float32)
    pe[:, 0::2] = np.sin(position * div_term)
    pe[:, 1::2] = np.cos(position * div_term)
    return pe


_PE = _make_pe()
_SCALE = math.sqrt(D_MODEL)


@functools.partial(
    pl.kernel,
    mesh=plsc.VectorSubcoreMesh(core_axis_name="c", subcore_axis_name="s"),
    out_type=jax.ShapeDtypeStruct((ROWS, D_MODEL), jnp.float32),
    compiler_params=pltpu.CompilerParams(use_tc_tiling_on_sc=False),
    scratch_types=[
        pltpu.VMEM((ROWS_PER_W // HALF, HALF), jnp.int32),
        pltpu.VMEM((SEQ, D_MODEL), jnp.float32),
        pltpu.VMEM((CHUNK, D_MODEL), jnp.float32),
        pltpu.VMEM((CHUNK, D_MODEL), jnp.float32),
        pltpu.SemaphoreType.DMA,
        pltpu.SemaphoreType.DMA,
        pltpu.SemaphoreType.DMA,
        pltpu.SemaphoreType.DMA,
    ],
)
def _sc_embed(x_hbm, pe_hbm, table_hbm, out_hbm,
              idx_v, pe_v, rows0, rows1, g0, g1, o0, o1):
    wid = lax.axis_index("s") * NC + lax.axis_index("c")
    rows = (rows0, rows1)
    gsem = (g0, g1)
    osem = (o0, o1)

    pltpu.sync_copy(x_hbm.at[wid], idx_v)
    pltpu.sync_copy(pe_hbm, pe_v)

    def start_gather(c, b):
        h0 = pltpu.async_copy(
            table_hbm.at[idx_v.at[2 * c]], rows[b].at[pl.ds(0, HALF)], gsem[b])
        h1 = pltpu.async_copy(
            table_hbm.at[idx_v.at[2 * c + 1]], rows[b].at[pl.ds(HALF, HALF)],
            gsem[b])
        return (h0, h1)

    def compute(b):
        rbuf = rows[b]

        def row(j, carry):
            for db in range(D_MODEL // 16):
                sl = pl.ds(db * 16, 16)
                rbuf[j, sl] = rbuf[j, sl] * _SCALE + pe_v[j, sl]
            return carry

        lax.fori_loop(0, CHUNK, row, 0, unroll=2)

    out_handles = [None, None]
    gather_handles = [None, None]

    gather_handles[0] = start_gather(0, 0)
    for c in range(NCHUNK):
        b = c % 2
        nb = (c + 1) % 2
        if c + 1 < NCHUNK:
            if out_handles[nb] is not None:
                out_handles[nb].wait()
                out_handles[nb] = None
            gather_handles[nb] = start_gather(c + 1, nb)
        for h in gather_handles[b]:
            h.wait()
        compute(b)
        base = wid * ROWS_PER_W + c * CHUNK
        out_handles[b] = pltpu.async_copy(
            rows[b], out_hbm.at[pl.ds(base, CHUNK)], osem[b])
    for h in out_handles:
        if h is not None:
            h.wait()


def kernel(x, emb_table):
    x_slab = x.reshape(NW, ROWS_PER_W // HALF, HALF)
    out = _sc_embed(x_slab, jnp.asarray(_PE), emb_table)
    return out.reshape(BATCH, SEQ, D_MODEL)


# speedup vs baseline: 1.5076x; 1.5076x over previous
import functools
import math

import jax
import jax.numpy as jnp
import numpy as np
from jax import lax
from jax.experimental import pallas as pl
from jax.experimental.pallas import tpu as pltpu
from jax.experimental.pallas import tpu_sc as plsc

D_MODEL = 64
VOCAB = 1000000
BATCH = 1024
SEQ = 200

NC = 2
NS = 16
NW = NC * NS

B_PER_W = BATCH // NW
NGRP = SEQ // 16


def _make_pe():
    position = np.arange(0, SEQ, dtype=np.float32)[:, None]
    div_term = np.exp(
        np.arange(0, D_MODEL, 2, dtype=np.float32) * -(math.log(10000.0) / D_MODEL)
    )
    pe = np.zeros((SEQ, D_MODEL), dtype=np.float32)
    pe[:, 0::2] = np.sin(position * div_term)
    pe[:, 1::2] = np.cos(position * div_term)
    return pe


_PE = _make_pe()
_SCALE = math.sqrt(D_MODEL)


@functools.partial(
    pl.kernel,
    mesh=plsc.VectorSubcoreMesh(core_axis_name="c", subcore_axis_name="s"),
    out_type=jax.ShapeDtypeStruct((BATCH * SEQ, D_MODEL), jnp.float32),
    scratch_types=[
        pltpu.VMEM((SEQ, D_MODEL), jnp.float32),
        pltpu.VMEM((SEQ,), jnp.int32),
        pltpu.VMEM((SEQ,), jnp.int32),
        pltpu.VMEM((SEQ, D_MODEL), jnp.float32),
        pltpu.VMEM((SEQ, D_MODEL), jnp.float32),
        pltpu.SemaphoreType.DMA,
        pltpu.SemaphoreType.DMA,
        pltpu.SemaphoreType.DMA,
        pltpu.SemaphoreType.DMA,
        pltpu.SemaphoreType.DMA,
        pltpu.SemaphoreType.DMA,
    ],
)
def _sc_embed(x_hbm, pe_hbm, table_hbm, out_hbm,
              pe_v, idx0, idx1, rows0, rows1,
              i0, i1, g0, g1, o0, o1):
    wid = lax.axis_index("s") * NC + lax.axis_index("c")
    idx = (idx0, idx1)
    rows = (rows0, rows1)
    isem = (i0, i1)
    gsem = (g0, g1)
    osem = (o0, o1)

    pltpu.sync_copy(pe_hbm, pe_v)

    def start_idx(j, p):
        pltpu.async_copy(x_hbm.at[wid * B_PER_W + j], idx[p], isem[p])

    def wait_idx(p):
        pltpu.make_async_copy(x_hbm.at[0], idx[p], isem[p]).wait()

    def enqueue_gathers(p):
        def grp(g, carry):
            v = idx[p][pl.ds(g * 16, 16)]
            base = g * 16
            for t in range(16):
                pltpu.async_copy(
                    table_hbm.at[v[t]], rows[p].at[base + t], gsem[p])
            return carry

        lax.fori_loop(0, NGRP, grp, 0)
        v = idx[p][pl.ds(SEQ - 16, 16)]
        for t in range(8, 16):
            pltpu.async_copy(
                table_hbm.at[v[t]], rows[p].at[SEQ - 16 + t], gsem[p])

    def wait_gathers(p):
        pltpu.make_async_copy(
            table_hbm.at[pl.ds(0, SEQ)], rows[p], gsem[p]).wait()

    def compute(p):
        rbuf = rows[p]

        def row(j, carry):
            for db in range(D_MODEL // 16):
                sl = pl.ds(db * 16, 16)
                rbuf[j, sl] = rbuf[j, sl] * _SCALE + pe_v[j, sl]
            return carry

        lax.fori_loop(0, SEQ, row, 0, unroll=2)

    def start_out(j, p):
        pltpu.async_copy(
            rows[p], out_hbm.at[pl.ds((wid * B_PER_W + j) * SEQ, SEQ)], osem[p])

    def wait_out(p):
        pltpu.make_async_copy(
            rows[p], out_hbm.at[pl.ds(0, SEQ)], osem[p]).wait()

    start_idx(0, 0)
    start_idx(1, 1)
    wait_idx(0)
    enqueue_gathers(0)
    start_idx(2, 0)

    def step(k, carry):
        b = k % 2
        nb = 1 - b

        def for_parity(b, nb):
            @pl.when(k >= 1)
            def _():
                wait_out(nb)

            @pl.when(k + 1 < B_PER_W)
            def _():
                wait_idx(nb)
                enqueue_gathers(nb)

            @pl.when(k + 3 < B_PER_W)
            def _():
                start_idx(k + 3, nb)

            wait_gathers(b)
            compute(b)
            start_out(k, b)

        @pl.when(b == 0)
        def _():
            for_parity(0, 1)

        @pl.when(b == 1)
        def _():
            for_parity(1, 0)

        return carry

    lax.fori_loop(0, B_PER_W, step, 0)
    wait_out((B_PER_W - 1) % 2)


def kernel(x, emb_table):
    out = _sc_embed(x, jnp.asarray(_PE), emb_table)
    return out.reshape(BATCH, SEQ, D_MODEL)


# speedup vs baseline: 1.5139x; 1.0041x over previous
import functools
import math

import jax
import jax.numpy as jnp
import numpy as np
from jax import lax
from jax.experimental import pallas as pl
from jax.experimental.pallas import tpu as pltpu
from jax.experimental.pallas import tpu_sc as plsc

D_MODEL = 64
VOCAB = 1000000
BATCH = 1024
SEQ = 200

NC = 2
NS = 16
NW = NC * NS

B_PER_W = BATCH // NW
NGRP = SEQ // 16


def _make_pe():
    position = np.arange(0, SEQ, dtype=np.float32)[:, None]
    div_term = np.exp(
        np.arange(0, D_MODEL, 2, dtype=np.float32) * -(math.log(10000.0) / D_MODEL)
    )
    pe = np.zeros((SEQ, D_MODEL), dtype=np.float32)
    pe[:, 0::2] = np.sin(position * div_term)
    pe[:, 1::2] = np.cos(position * div_term)
    return pe


_PE = _make_pe()
_SCALE = math.sqrt(D_MODEL)


@functools.partial(
    pl.kernel,
    mesh=plsc.VectorSubcoreMesh(core_axis_name="c", subcore_axis_name="s"),
    out_type=jax.ShapeDtypeStruct((BATCH * SEQ, D_MODEL), jnp.float32),
    compiler_params=pltpu.CompilerParams(use_tc_tiling_on_sc=True),
    scratch_types=[
        pltpu.VMEM((SEQ, D_MODEL), jnp.float32),
        pltpu.VMEM((SEQ,), jnp.int32),
        pltpu.VMEM((SEQ,), jnp.int32),
        pltpu.VMEM((SEQ, D_MODEL), jnp.float32),
        pltpu.VMEM((SEQ, D_MODEL), jnp.float32),
        pltpu.SemaphoreType.DMA,
        pltpu.SemaphoreType.DMA,
        pltpu.SemaphoreType.DMA,
        pltpu.SemaphoreType.DMA,
        pltpu.SemaphoreType.DMA,
        pltpu.SemaphoreType.DMA,
    ],
)
def _sc_embed(x_hbm, pe_hbm, table_hbm, out_hbm,
              pe_v, idx0, idx1, rows0, rows1,
              i0, i1, g0, g1, o0, o1):
    wid = lax.axis_index("s") * NC + lax.axis_index("c")
    idx = (idx0, idx1)
    rows = (rows0, rows1)
    isem = (i0, i1)
    gsem = (g0, g1)
    osem = (o0, o1)

    pltpu.sync_copy(pe_hbm, pe_v)

    def start_idx(j, p):
        pltpu.async_copy(x_hbm.at[wid * B_PER_W + j], idx[p], isem[p])

    def wait_idx(p):
        pltpu.make_async_copy(x_hbm.at[0], idx[p], isem[p]).wait()

    def enqueue_gathers(p):
        def grp(g, carry):
            v = idx[p][pl.ds(g * 16, 16)]
            base = g * 16
            for t in range(16):
                pltpu.async_copy(
                    table_hbm.at[v[t]], rows[p].at[base + t], gsem[p])
            return carry

        lax.fori_loop(0, NGRP, grp, 0)
        v = idx[p][pl.ds(SEQ - 16, 16)]
        for t in range(8, 16):
            pltpu.async_copy(
                table_hbm.at[v[t]], rows[p].at[SEQ - 16 + t], gsem[p])

    def wait_gathers(p):
        pltpu.make_async_copy(
            table_hbm.at[pl.ds(0, SEQ)], rows[p], gsem[p]).wait()

    def compute(p):
        rbuf = rows[p]

        def row(j, carry):
            for db in range(D_MODEL // 16):
                sl = pl.ds(db * 16, 16)
                rbuf[j, sl] = rbuf[j, sl] * _SCALE + pe_v[j, sl]
            return carry

        lax.fori_loop(0, SEQ, row, 0, unroll=2)

    def start_out(j, p):
        pltpu.async_copy(
            rows[p], out_hbm.at[pl.ds((wid * B_PER_W + j) * SEQ, SEQ)], osem[p])

    def wait_out(p):
        pltpu.make_async_copy(
            rows[p], out_hbm.at[pl.ds(0, SEQ)], osem[p]).wait()

    start_idx(0, 0)
    start_idx(1, 1)
    wait_idx(0)
    enqueue_gathers(0)
    start_idx(2, 0)

    def step(k, carry):
        b = k % 2
        nb = 1 - b

        def for_parity(b, nb):
            @pl.when(k >= 1)
            def _():
                wait_out(nb)

            @pl.when(k + 1 < B_PER_W)
            def _():
                wait_idx(nb)
                enqueue_gathers(nb)

            @pl.when(k + 3 < B_PER_W)
            def _():
                start_idx(k + 3, nb)

            wait_gathers(b)
            compute(b)
            start_out(k, b)

        @pl.when(b == 0)
        def _():
            for_parity(0, 1)

        @pl.when(b == 1)
        def _():
            for_parity(1, 0)

        return carry

    lax.fori_loop(0, B_PER_W, step, 0)
    wait_out((B_PER_W - 1) % 2)


def kernel(x, emb_table):
    out = _sc_embed(x, jnp.asarray(_PE), emb_table)
    return out.reshape(BATCH, SEQ, D_MODEL)
